# baseline (speedup 1.0000x reference)
import functools

import jax
import jax.numpy as jnp
from jax import lax
from jax.experimental import pallas as pl
from jax.experimental.pallas import tpu as pltpu
from jax.experimental.pallas import tpu_sc as plsc

N = 10000
D = 128
C = 40

NC = 2
NS = 16
BLK = 128
TBL = 10240
RPT = TBL // NS
ICH = 16

_EPS = 1e-12


def _rownorm_kernel(x_ref, o_ref):
    v = x_ref[...]
    n = jnp.sqrt(jnp.sum(v * v, axis=1, keepdims=True))
    o_ref[...] = v / jnp.maximum(n, _EPS)


def _rownorm(x):
    rb = lambda i: (i, 0)
    return pl.pallas_call(
        _rownorm_kernel,
        grid=(N // 2000,),
        in_specs=[pl.BlockSpec((2000, D), rb)],
        out_specs=pl.BlockSpec((2000, D), rb),
        out_shape=jax.ShapeDtypeStruct((TBL, D), x.dtype),
    )(x)


def _segsum_partials(xnp, src2d, dst2d, zeros):
    eb = src2d.shape[0]
    bpt = eb // (NC * NS)
    mesh = plsc.VectorSubcoreMesh(core_axis_name="c", subcore_axis_name="s")

    @functools.partial(
        pl.kernel,
        out_type=(
            jax.ShapeDtypeStruct((TBL, D), jnp.float32),
            jax.ShapeDtypeStruct((TBL, D), jnp.float32),
            jax.ShapeDtypeStruct((eb * BLK, D), jnp.float32),
        ),
        mesh=mesh,
        scratch_types=[
            pltpu.VMEM((ICH, BLK), jnp.int32),
            pltpu.VMEM((ICH, BLK), jnp.int32),
            pltpu.VMEM((BLK, D), jnp.float32),
            pltpu.VMEM((BLK, D), jnp.float32),
            pltpu.VMEM_SHARED((TBL, D), jnp.float32),
            pltpu.SemaphoreType.DMA,
            pltpu.SemaphoreType.DMA,
            pltpu.SemaphoreType.DMA,
            pltpu.SemaphoreType.DMA,
        ],
    )
    def k(xn_hbm, src_hbm, dst_hbm, z_hbm, out0_hbm, out1_hbm, msgs_hbm,
          src_c, dst_c, g0, g1, table, sem0, sem1, sem2, sem3):
        c = lax.axis_index("c")
        s = lax.axis_index("s")
        wid = s * NC + c

        pltpu.sync_copy(xn_hbm.at[pl.ds(s * RPT, RPT)],
                        table.at[pl.ds(s * RPT, RPT)])
        plsc.subcore_barrier()

        @pl.loop(0, bpt // ICH)
        def _(g):
            base = wid * bpt + g * ICH
            pltpu.sync_copy(src_hbm.at[pl.ds(base, ICH)], src_c)
            pltpu.async_copy(table.at[src_c.at[0]], g0, sem0)
            pltpu.async_copy(table.at[src_c.at[1]], g1, sem1)

            @pl.loop(0, ICH // 2)
            def _(i):
                j = i * 2
                pltpu.make_async_copy(table.at[src_c.at[j]], g0, sem0).wait()
                pltpu.async_copy(
                    g0, msgs_hbm.at[pl.ds((base + j) * BLK, BLK)], sem2)
                pltpu.make_async_copy(
                    table.at[src_c.at[j + 1]], g1, sem1).wait()
                pltpu.async_copy(
                    g1, msgs_hbm.at[pl.ds((base + j + 1) * BLK, BLK)], sem3)
                pltpu.make_async_copy(
                    g0, msgs_hbm.at[pl.ds((base + j) * BLK, BLK)],
                    sem2).wait()

                @pl.when(j + 2 < ICH)
                def _():
                    pltpu.async_copy(table.at[src_c.at[j + 2]], g0, sem0)

                pltpu.make_async_copy(
                    g1, msgs_hbm.at[pl.ds((base + j + 1) * BLK, BLK)],
                    sem3).wait()

                @pl.when(j + 3 < ICH)
                def _():
                    pltpu.async_copy(table.at[src_c.at[j + 3]], g1, sem1)

        plsc.subcore_barrier()

        @pl.loop(0, RPT // 64)
        def _(z):
            pltpu.sync_copy(z_hbm, table.at[pl.ds(s * RPT + z * 64, 64)])
        plsc.subcore_barrier()

        @pl.loop(0, bpt // ICH)
        def _(g):
            base = wid * bpt + g * ICH
            pltpu.sync_copy(dst_hbm.at[pl.ds(base, ICH)], dst_c)
            pltpu.async_copy(msgs_hbm.at[pl.ds(base * BLK, BLK)], g0, sem0)
            pltpu.async_copy(msgs_hbm.at[pl.ds((base + 1) * BLK, BLK)], g1,
                             sem1)

            @pl.loop(0, ICH // 2)
            def _(i):
                j = i * 2
                pltpu.make_async_copy(
                    msgs_hbm.at[pl.ds((base + j) * BLK, BLK)], g0,
                    sem0).wait()
                pltpu.async_copy(g0, table.at[dst_c.at[j]], sem2, add=True)
                pltpu.make_async_copy(
                    msgs_hbm.at[pl.ds((base + j + 1) * BLK, BLK)], g1,
                    sem1).wait()
                pltpu.async_copy(g1, table.at[dst_c.at[j + 1]], sem3,
                                 add=True)
                pltpu.make_async_copy(g0, table.at[dst_c.at[j]], sem2).wait()

                @pl.when(j + 2 < ICH)
                def _():
                    pltpu.async_copy(
                        msgs_hbm.at[pl.ds((base + j + 2) * BLK, BLK)], g0,
                        sem0)

                pltpu.make_async_copy(g1, table.at[dst_c.at[j + 1]],
                                      sem3).wait()

                @pl.when(j + 3 < ICH)
                def _():
                    pltpu.async_copy(
                        msgs_hbm.at[pl.ds((base + j + 3) * BLK, BLK)], g1,
                        sem1)

        plsc.subcore_barrier()

        @pl.when(c == 0)
        def _():
            pltpu.sync_copy(table.at[pl.ds(s * RPT, RPT)],
                            out0_hbm.at[pl.ds(s * RPT, RPT)])

        @pl.when(c == 1)
        def _():
            pltpu.sync_copy(table.at[pl.ds(s * RPT, RPT)],
                            out1_hbm.at[pl.ds(s * RPT, RPT)])

    p0, p1, _ = k(xnp, src2d, dst2d, zeros)
    return p0, p1


def _dotT(a, w):
    return lax.dot_general(
        a, w, dimension_numbers=(((1,), (1,)), ((), ())),
        preferred_element_type=jnp.float32,
        precision=lax.Precision.HIGHEST)


def _layer_mid_kernel(xn_ref, p0_ref, p1_ref, x_ref, wl_ref, bl_ref, wr_ref,
                      sc_ref, h_ref, hn_ref):
    xn = xn_ref[...]
    x = x_ref[...]
    agg = xn + p0_ref[...] + p1_ref[...]
    nagg = jnp.sqrt(jnp.sum(agg * agg, axis=1, keepdims=True))
    aggn = agg / jnp.maximum(nagg, _EPS)
    xnorm = jnp.sqrt(jnp.sum(x * x, axis=1, keepdims=True))
    m = aggn * xnorm * sc_ref[0, 0]
    o = _dotT(m, wl_ref[...]) + bl_ref[...] + _dotT(x, wr_ref[...])
    no = jnp.sqrt(jnp.sum(o * o, axis=1, keepdims=True))
    h = jnp.maximum(o / jnp.maximum(no, _EPS), 0.0)
    h_ref[...] = h
    nh = jnp.sqrt(jnp.sum(h * h, axis=1, keepdims=True))
    hn_ref[...] = h / jnp.maximum(nh, _EPS)


def _layer_mid(xn, p0, p1, x, wl, bl, wr, sc):
    grid = (N // 2000,)
    rb = lambda i: (i, 0)
    wb = lambda i: (0, 0)
    return pl.pallas_call(
        _layer_mid_kernel,
        grid=grid,
        in_specs=[
            pl.BlockSpec((2000, D), rb),
            pl.BlockSpec((2000, D), rb),
            pl.BlockSpec((2000, D), rb),
            pl.BlockSpec((2000, D), rb),
            pl.BlockSpec((D, D), wb),
            pl.BlockSpec((1, D), wb),
            pl.BlockSpec((D, D), wb),
            pl.BlockSpec((1, 1), wb),
        ],
        out_specs=[
            pl.BlockSpec((2000, D), rb),
            pl.BlockSpec((2000, D), rb),
        ],
        out_shape=[
            jax.ShapeDtypeStruct((N, D), jnp.float32),
            jax.ShapeDtypeStruct((TBL, D), jnp.float32),
        ],
    )(xn, p0, p1, x, wl, bl, wr, sc)


def _layer_final_kernel(hn_ref, p0_ref, p1_ref, h_ref, wl_ref, bl_ref, wr_ref,
                        sc_ref, o_ref):
    hn = hn_ref[...]
    h = h_ref[...]
    agg = hn + p0_ref[...] + p1_ref[...]
    nagg = jnp.sqrt(jnp.sum(agg * agg, axis=1, keepdims=True))
    aggn = agg / jnp.maximum(nagg, _EPS)
    hnorm = jnp.sqrt(jnp.sum(h * h, axis=1, keepdims=True))
    m = aggn * hnorm * sc_ref[0, 0]
    o = _dotT(m, wl_ref[...]) + bl_ref[...] + _dotT(h, wr_ref[...])
    no = jnp.sqrt(jnp.sum(o * o, axis=1, keepdims=True))
    o_ref[...] = o / jnp.maximum(no, _EPS)


def _layer_final(hn, p0, p1, h, wl, bl, wr, sc):
    grid = (N // 2000,)
    rb = lambda i: (i, 0)
    wb = lambda i: (0, 0)
    return pl.pallas_call(
        _layer_final_kernel,
        grid=grid,
        in_specs=[
            pl.BlockSpec((2000, D), rb),
            pl.BlockSpec((2000, D), rb),
            pl.BlockSpec((2000, D), rb),
            pl.BlockSpec((2000, D), rb),
            pl.BlockSpec((C, D), wb),
            pl.BlockSpec((1, C), wb),
            pl.BlockSpec((C, D), wb),
            pl.BlockSpec((1, 1), wb),
        ],
        out_specs=pl.BlockSpec((2000, C), rb),
        out_shape=jax.ShapeDtypeStruct((N, C), jnp.float32),
    )(hn, p0, p1, h, wl, bl, wr, sc)


def kernel(x, edge_index, Wl0, bl0, Wr0, scale0, Wl1, bl1, Wr1, scale1):
    src = edge_index[0]
    dst = edge_index[1]
    e = src.shape[0]
    chunk = NC * NS * BLK * 8
    epad = ((e + chunk - 1) // chunk) * chunk
    pad = epad - e
    src_p = jnp.concatenate([src, jnp.zeros((pad,), jnp.int32)])
    dst_p = jnp.concatenate([dst, jnp.full((pad,), N, jnp.int32)])
    src2d = src_p.reshape(-1, BLK)
    dst2d = dst_p.reshape(-1, BLK)
    zeros = jnp.zeros((64, D), jnp.float32)
    bl0r = bl0.reshape(1, D)
    bl1r = bl1.reshape(1, C)
    s0r = scale0.reshape(1, 1)
    s1r = scale1.reshape(1, 1)

    xn = _rownorm(x)
    p0, p1 = _segsum_partials(xn, src2d, dst2d, zeros)
    h, hn = _layer_mid(xn, p0, p1, x, Wl0, bl0r, Wr0, s0r)
    q0, q1 = _segsum_partials(hn, src2d, dst2d, zeros)
    out = _layer_final(hn, q0, q1, h, Wl1, bl1r, Wr1, s1r)
    return out


# speedup vs baseline: 1.1934x; 1.1934x over previous
import functools

import jax
import jax.numpy as jnp
from jax import lax
from jax.experimental import pallas as pl
from jax.experimental.pallas import tpu as pltpu
from jax.experimental.pallas import tpu_sc as plsc

N = 10000
D = 128
C = 40

NC = 2
NS = 16
BLK = 128
TBL = 10240
RPT = TBL // NS
ICH = 16

_EPS = 1e-12


def _rownorm_kernel(x_ref, o_ref):
    v = x_ref[...]
    n = jnp.sqrt(jnp.sum(v * v, axis=1, keepdims=True))
    o_ref[...] = v / jnp.maximum(n, _EPS)


def _rownorm(x):
    rb = lambda i: (i, 0)
    return pl.pallas_call(
        _rownorm_kernel,
        grid=(N // 2000,),
        in_specs=[pl.BlockSpec((2000, D), rb)],
        out_specs=pl.BlockSpec((2000, D), rb),
        out_shape=jax.ShapeDtypeStruct((TBL, D), x.dtype),
    )(x)


def _segsum_partials(xnp, src2d, dst2d, zeros):
    eb = src2d.shape[0]
    bpt = eb // (NC * NS)
    mesh = plsc.VectorSubcoreMesh(core_axis_name="c", subcore_axis_name="s")

    @functools.partial(
        pl.kernel,
        out_type=(
            jax.ShapeDtypeStruct((TBL, D), jnp.float32),
            jax.ShapeDtypeStruct((TBL, D), jnp.float32),
            jax.ShapeDtypeStruct((eb * BLK, D), jnp.float32),
        ),
        mesh=mesh,
        scratch_types=[
            pltpu.VMEM((ICH, BLK), jnp.int32),
            pltpu.VMEM((ICH, BLK), jnp.int32),
            pltpu.VMEM((BLK, D), jnp.float32),
            pltpu.VMEM((BLK, D), jnp.float32),
            pltpu.VMEM_SHARED((TBL, D), jnp.float32),
            pltpu.SemaphoreType.DMA,
            pltpu.SemaphoreType.DMA,
            pltpu.SemaphoreType.DMA,
            pltpu.SemaphoreType.DMA,
        ],
    )
    def k(xn_hbm, src_hbm, dst_hbm, z_hbm, out0_hbm, out1_hbm, msgs_hbm,
          src_c, dst_c, g0, g1, table, sem0, sem1, sem2, sem3):
        c = lax.axis_index("c")
        s = lax.axis_index("s")
        wid = s * NC + c

        pltpu.sync_copy(xn_hbm.at[pl.ds(s * RPT, RPT)],
                        table.at[pl.ds(s * RPT, RPT)])
        plsc.subcore_barrier()

        @pl.loop(0, bpt // ICH)
        def _(g):
            base = wid * bpt + g * ICH
            pltpu.sync_copy(src_hbm.at[pl.ds(base, ICH)], src_c)
            pltpu.sync_copy(table.at[src_c.at[0]], g0)

            @pl.loop(0, ICH // 2)
            def _(i):
                j = i * 2
                pltpu.async_copy(
                    g0, msgs_hbm.at[pl.ds((base + j) * BLK, BLK)], sem0)
                pltpu.sync_copy(table.at[src_c.at[j + 1]], g1)
                pltpu.make_async_copy(
                    g0, msgs_hbm.at[pl.ds((base + j) * BLK, BLK)],
                    sem0).wait()
                pltpu.async_copy(
                    g1, msgs_hbm.at[pl.ds((base + j + 1) * BLK, BLK)], sem1)

                @pl.when(j + 2 < ICH)
                def _():
                    pltpu.sync_copy(table.at[src_c.at[j + 2]], g0)

                pltpu.make_async_copy(
                    g1, msgs_hbm.at[pl.ds((base + j + 1) * BLK, BLK)],
                    sem1).wait()

        plsc.subcore_barrier()

        @pl.loop(0, RPT // 64)
        def _(z):
            pltpu.sync_copy(z_hbm, table.at[pl.ds(s * RPT + z * 64, 64)])
        plsc.subcore_barrier()

        @pl.loop(0, bpt // ICH)
        def _(g):
            base = wid * bpt + g * ICH
            pltpu.sync_copy(dst_hbm.at[pl.ds(base, ICH)], dst_c)
            pltpu.async_copy(msgs_hbm.at[pl.ds(base * BLK, BLK)], g0, sem0)

            @pl.loop(0, ICH // 2)
            def _(i):
                j = i * 2
                pltpu.make_async_copy(
                    msgs_hbm.at[pl.ds((base + j) * BLK, BLK)], g0,
                    sem0).wait()
                pltpu.async_copy(
                    msgs_hbm.at[pl.ds((base + j + 1) * BLK, BLK)], g1, sem1)
                pltpu.sync_copy(g0, table.at[dst_c.at[j]], add=True)
                pltpu.make_async_copy(
                    msgs_hbm.at[pl.ds((base + j + 1) * BLK, BLK)], g1,
                    sem1).wait()

                @pl.when(j + 2 < ICH)
                def _():
                    pltpu.async_copy(
                        msgs_hbm.at[pl.ds((base + j + 2) * BLK, BLK)], g0,
                        sem0)

                pltpu.sync_copy(g1, table.at[dst_c.at[j + 1]], add=True)

        plsc.subcore_barrier()

        @pl.when(c == 0)
        def _():
            pltpu.sync_copy(table.at[pl.ds(s * RPT, RPT)],
                            out0_hbm.at[pl.ds(s * RPT, RPT)])

        @pl.when(c == 1)
        def _():
            pltpu.sync_copy(table.at[pl.ds(s * RPT, RPT)],
                            out1_hbm.at[pl.ds(s * RPT, RPT)])

    p0, p1, _ = k(xnp, src2d, dst2d, zeros)
    return p0, p1


def _dotT(a, w):
    return lax.dot_general(
        a, w, dimension_numbers=(((1,), (1,)), ((), ())),
        preferred_element_type=jnp.float32,
        precision=lax.Precision.HIGHEST)


def _layer_mid_kernel(xn_ref, p0_ref, p1_ref, x_ref, wl_ref, bl_ref, wr_ref,
                      sc_ref, h_ref, hn_ref):
    xn = xn_ref[...]
    x = x_ref[...]
    agg = xn + p0_ref[...] + p1_ref[...]
    nagg = jnp.sqrt(jnp.sum(agg * agg, axis=1, keepdims=True))
    aggn = agg / jnp.maximum(nagg, _EPS)
    xnorm = jnp.sqrt(jnp.sum(x * x, axis=1, keepdims=True))
    m = aggn * xnorm * sc_ref[0, 0]
    o = _dotT(m, wl_ref[...]) + bl_ref[...] + _dotT(x, wr_ref[...])
    no = jnp.sqrt(jnp.sum(o * o, axis=1, keepdims=True))
    h = jnp.maximum(o / jnp.maximum(no, _EPS), 0.0)
    h_ref[...] = h
    nh = jnp.sqrt(jnp.sum(h * h, axis=1, keepdims=True))
    hn_ref[...] = h / jnp.maximum(nh, _EPS)


def _layer_mid(xn, p0, p1, x, wl, bl, wr, sc):
    grid = (N // 2000,)
    rb = lambda i: (i, 0)
    wb = lambda i: (0, 0)
    return pl.pallas_call(
        _layer_mid_kernel,
        grid=grid,
        in_specs=[
            pl.BlockSpec((2000, D), rb),
            pl.BlockSpec((2000, D), rb),
            pl.BlockSpec((2000, D), rb),
            pl.BlockSpec((2000, D), rb),
            pl.BlockSpec((D, D), wb),
            pl.BlockSpec((1, D), wb),
            pl.BlockSpec((D, D), wb),
            pl.BlockSpec((1, 1), wb),
        ],
        out_specs=[
            pl.BlockSpec((2000, D), rb),
            pl.BlockSpec((2000, D), rb),
        ],
        out_shape=[
            jax.ShapeDtypeStruct((N, D), jnp.float32),
            jax.ShapeDtypeStruct((TBL, D), jnp.float32),
        ],
    )(xn, p0, p1, x, wl, bl, wr, sc)


def _layer_final_kernel(hn_ref, p0_ref, p1_ref, h_ref, wl_ref, bl_ref, wr_ref,
                        sc_ref, o_ref):
    hn = hn_ref[...]
    h = h_ref[...]
    agg = hn + p0_ref[...] + p1_ref[...]
    nagg = jnp.sqrt(jnp.sum(agg * agg, axis=1, keepdims=True))
    aggn = agg / jnp.maximum(nagg, _EPS)
    hnorm = jnp.sqrt(jnp.sum(h * h, axis=1, keepdims=True))
    m = aggn * hnorm * sc_ref[0, 0]
    o = _dotT(m, wl_ref[...]) + bl_ref[...] + _dotT(h, wr_ref[...])
    no = jnp.sqrt(jnp.sum(o * o, axis=1, keepdims=True))
    o_ref[...] = o / jnp.maximum(no, _EPS)


def _layer_final(hn, p0, p1, h, wl, bl, wr, sc):
    grid = (N // 2000,)
    rb = lambda i: (i, 0)
    wb = lambda i: (0, 0)
    return pl.pallas_call(
        _layer_final_kernel,
        grid=grid,
        in_specs=[
            pl.BlockSpec((2000, D), rb),
            pl.BlockSpec((2000, D), rb),
            pl.BlockSpec((2000, D), rb),
            pl.BlockSpec((2000, D), rb),
            pl.BlockSpec((C, D), wb),
            pl.BlockSpec((1, C), wb),
            pl.BlockSpec((C, D), wb),
            pl.BlockSpec((1, 1), wb),
        ],
        out_specs=pl.BlockSpec((2000, C), rb),
        out_shape=jax.ShapeDtypeStruct((N, C), jnp.float32),
    )(hn, p0, p1, h, wl, bl, wr, sc)


def kernel(x, edge_index, Wl0, bl0, Wr0, scale0, Wl1, bl1, Wr1, scale1):
    src = edge_index[0]
    dst = edge_index[1]
    e = src.shape[0]
    chunk = NC * NS * BLK * 8
    epad = ((e + chunk - 1) // chunk) * chunk
    pad = epad - e
    src_p = jnp.concatenate([src, jnp.zeros((pad,), jnp.int32)])
    dst_p = jnp.concatenate([dst, jnp.full((pad,), N, jnp.int32)])
    src2d = src_p.reshape(-1, BLK)
    dst2d = dst_p.reshape(-1, BLK)
    zeros = jnp.zeros((64, D), jnp.float32)
    bl0r = bl0.reshape(1, D)
    bl1r = bl1.reshape(1, C)
    s0r = scale0.reshape(1, 1)
    s1r = scale1.reshape(1, 1)

    xn = _rownorm(x)
    p0, p1 = _segsum_partials(xn, src2d, dst2d, zeros)
    h, hn = _layer_mid(xn, p0, p1, x, Wl0, bl0r, Wr0, s0r)
    q0, q1 = _segsum_partials(hn, src2d, dst2d, zeros)
    out = _layer_final(hn, q0, q1, h, Wl1, bl1r, Wr1, s1r)
    return out


# speedup vs baseline: 1.2334x; 1.0335x over previous
import functools

import jax
import jax.numpy as jnp
from jax import lax
from jax.experimental import pallas as pl
from jax.experimental.pallas import tpu as pltpu
from jax.experimental.pallas import tpu_sc as plsc

N = 10000
D = 128
C = 40

NC = 2
NS = 16
BLK = 128
TBL = 10240
RPT = TBL // NS

_EPS = 1e-12


def _rownorm_kernel(x_ref, o_ref):
    v = x_ref[...]
    n = jnp.sqrt(jnp.sum(v * v, axis=1, keepdims=True))
    o_ref[...] = v / jnp.maximum(n, _EPS)


def _rownorm(x):
    rb = lambda i: (i, 0)
    return pl.pallas_call(
        _rownorm_kernel,
        grid=(N // 2000,),
        in_specs=[pl.BlockSpec((2000, D), rb)],
        out_specs=pl.BlockSpec((2000, D), rb),
        out_shape=jax.ShapeDtypeStruct((TBL, D), x.dtype),
    )(x)


def _segsum_partials(xnp, src2d, dst2d, zeros):
    eb = src2d.shape[0]
    bpt = eb // (NC * NS)
    mesh = plsc.VectorSubcoreMesh(core_axis_name="c", subcore_axis_name="s")

    @functools.partial(
        pl.kernel,
        out_type=(
            jax.ShapeDtypeStruct((TBL, D), jnp.float32),
            jax.ShapeDtypeStruct((TBL, D), jnp.float32),
            jax.ShapeDtypeStruct((eb * BLK, D), jnp.float32),
        ),
        mesh=mesh,
        scratch_types=[
            pltpu.VMEM((bpt, BLK), jnp.int32),
            pltpu.VMEM((BLK, D), jnp.float32),
            pltpu.VMEM((BLK, D), jnp.float32),
            pltpu.VMEM_SHARED((TBL, D), jnp.float32),
            pltpu.SemaphoreType.DMA,
            pltpu.SemaphoreType.DMA,
        ],
    )
    def k(xn_hbm, src_hbm, dst_hbm, z_hbm, out0_hbm, out1_hbm, msgs_hbm,
          idx_c, g0, g1, table, sem0, sem1):
        c = lax.axis_index("c")
        s = lax.axis_index("s")
        wid = s * NC + c
        base = wid * bpt

        pltpu.sync_copy(xn_hbm.at[pl.ds(s * RPT, RPT)],
                        table.at[pl.ds(s * RPT, RPT)])
        pltpu.sync_copy(src_hbm.at[pl.ds(base, bpt)], idx_c)
        plsc.subcore_barrier()
        pltpu.sync_copy(table.at[idx_c.at[0]], g0)

        @pl.loop(0, bpt // 2)
        def _(i):
            j = i * 2
            pltpu.async_copy(
                g0, msgs_hbm.at[pl.ds((base + j) * BLK, BLK)], sem0)
            pltpu.sync_copy(table.at[idx_c.at[j + 1]], g1)
            pltpu.make_async_copy(
                g0, msgs_hbm.at[pl.ds((base + j) * BLK, BLK)], sem0).wait()
            pltpu.async_copy(
                g1, msgs_hbm.at[pl.ds((base + j + 1) * BLK, BLK)], sem1)

            @pl.when(j + 2 < bpt)
            def _():
                pltpu.sync_copy(table.at[idx_c.at[j + 2]], g0)

            pltpu.make_async_copy(
                g1, msgs_hbm.at[pl.ds((base + j + 1) * BLK, BLK)],
                sem1).wait()

        plsc.subcore_barrier()

        @pl.loop(0, RPT // 64)
        def _(z):
            pltpu.sync_copy(z_hbm, table.at[pl.ds(s * RPT + z * 64, 64)])
        pltpu.sync_copy(dst_hbm.at[pl.ds(base, bpt)], idx_c)
        plsc.subcore_barrier()
        pltpu.async_copy(msgs_hbm.at[pl.ds(base * BLK, BLK)], g0, sem0)

        @pl.loop(0, bpt // 2)
        def _(i):
            j = i * 2
            pltpu.make_async_copy(
                msgs_hbm.at[pl.ds((base + j) * BLK, BLK)], g0, sem0).wait()
            pltpu.async_copy(
                msgs_hbm.at[pl.ds((base + j + 1) * BLK, BLK)], g1, sem1)
            pltpu.sync_copy(g0, table.at[idx_c.at[j]], add=True)
            pltpu.make_async_copy(
                msgs_hbm.at[pl.ds((base + j + 1) * BLK, BLK)], g1,
                sem1).wait()

            @pl.when(j + 2 < bpt)
            def _():
                pltpu.async_copy(
                    msgs_hbm.at[pl.ds((base + j + 2) * BLK, BLK)], g0, sem0)

            pltpu.sync_copy(g1, table.at[idx_c.at[j + 1]], add=True)

        plsc.subcore_barrier()

        @pl.when(c == 0)
        def _():
            pltpu.sync_copy(table.at[pl.ds(s * RPT, RPT)],
                            out0_hbm.at[pl.ds(s * RPT, RPT)])

        @pl.when(c == 1)
        def _():
            pltpu.sync_copy(table.at[pl.ds(s * RPT, RPT)],
                            out1_hbm.at[pl.ds(s * RPT, RPT)])

    p0, p1, _ = k(xnp, src2d, dst2d, zeros)
    return p0, p1


def _dotT(a, w):
    return lax.dot_general(
        a, w, dimension_numbers=(((1,), (1,)), ((), ())),
        preferred_element_type=jnp.float32,
        precision=lax.Precision.HIGHEST)


def _layer_mid_kernel(xn_ref, p0_ref, p1_ref, x_ref, wl_ref, bl_ref, wr_ref,
                      sc_ref, h_ref, hn_ref):
    xn = xn_ref[...]
    x = x_ref[...]
    agg = xn + p0_ref[...] + p1_ref[...]
    nagg = jnp.sqrt(jnp.sum(agg * agg, axis=1, keepdims=True))
    aggn = agg / jnp.maximum(nagg, _EPS)
    xnorm = jnp.sqrt(jnp.sum(x * x, axis=1, keepdims=True))
    m = aggn * xnorm * sc_ref[0, 0]
    o = _dotT(m, wl_ref[...]) + bl_ref[...] + _dotT(x, wr_ref[...])
    no = jnp.sqrt(jnp.sum(o * o, axis=1, keepdims=True))
    h = jnp.maximum(o / jnp.maximum(no, _EPS), 0.0)
    h_ref[...] = h
    nh = jnp.sqrt(jnp.sum(h * h, axis=1, keepdims=True))
    hn_ref[...] = h / jnp.maximum(nh, _EPS)


def _layer_mid(xn, p0, p1, x, wl, bl, wr, sc):
    grid = (N // 2000,)
    rb = lambda i: (i, 0)
    wb = lambda i: (0, 0)
    return pl.pallas_call(
        _layer_mid_kernel,
        grid=grid,
        in_specs=[
            pl.BlockSpec((2000, D), rb),
            pl.BlockSpec((2000, D), rb),
            pl.BlockSpec((2000, D), rb),
            pl.BlockSpec((2000, D), rb),
            pl.BlockSpec((D, D), wb),
            pl.BlockSpec((1, D), wb),
            pl.BlockSpec((D, D), wb),
            pl.BlockSpec((1, 1), wb),
        ],
        out_specs=[
            pl.BlockSpec((2000, D), rb),
            pl.BlockSpec((2000, D), rb),
        ],
        out_shape=[
            jax.ShapeDtypeStruct((N, D), jnp.float32),
            jax.ShapeDtypeStruct((TBL, D), jnp.float32),
        ],
    )(xn, p0, p1, x, wl, bl, wr, sc)


def _layer_final_kernel(hn_ref, p0_ref, p1_ref, h_ref, wl_ref, bl_ref, wr_ref,
                        sc_ref, o_ref):
    hn = hn_ref[...]
    h = h_ref[...]
    agg = hn + p0_ref[...] + p1_ref[...]
    nagg = jnp.sqrt(jnp.sum(agg * agg, axis=1, keepdims=True))
    aggn = agg / jnp.maximum(nagg, _EPS)
    hnorm = jnp.sqrt(jnp.sum(h * h, axis=1, keepdims=True))
    m = aggn * hnorm * sc_ref[0, 0]
    o = _dotT(m, wl_ref[...]) + bl_ref[...] + _dotT(h, wr_ref[...])
    no = jnp.sqrt(jnp.sum(o * o, axis=1, keepdims=True))
    o_ref[...] = o / jnp.maximum(no, _EPS)


def _layer_final(hn, p0, p1, h, wl, bl, wr, sc):
    grid = (N // 2000,)
    rb = lambda i: (i, 0)
    wb = lambda i: (0, 0)
    return pl.pallas_call(
        _layer_final_kernel,
        grid=grid,
        in_specs=[
            pl.BlockSpec((2000, D), rb),
            pl.BlockSpec((2000, D), rb),
            pl.BlockSpec((2000, D), rb),
            pl.BlockSpec((2000, D), rb),
            pl.BlockSpec((C, D), wb),
            pl.BlockSpec((1, C), wb),
            pl.BlockSpec((C, D), wb),
            pl.BlockSpec((1, 1), wb),
        ],
        out_specs=pl.BlockSpec((2000, C), rb),
        out_shape=jax.ShapeDtypeStruct((N, C), jnp.float32),
    )(hn, p0, p1, h, wl, bl, wr, sc)


def kernel(x, edge_index, Wl0, bl0, Wr0, scale0, Wl1, bl1, Wr1, scale1):
    src = edge_index[0]
    dst = edge_index[1]
    e = src.shape[0]
    chunk = NC * NS * BLK * 8
    epad = ((e + chunk - 1) // chunk) * chunk
    pad = epad - e
    src_p = jnp.concatenate([src, jnp.zeros((pad,), jnp.int32)])
    dst_p = jnp.concatenate([dst, jnp.full((pad,), N, jnp.int32)])
    src2d = src_p.reshape(-1, BLK)
    dst2d = dst_p.reshape(-1, BLK)
    zeros = jnp.zeros((64, D), jnp.float32)
    bl0r = bl0.reshape(1, D)
    bl1r = bl1.reshape(1, C)
    s0r = scale0.reshape(1, 1)
    s1r = scale1.reshape(1, 1)

    xn = _rownorm(x)
    p0, p1 = _segsum_partials(xn, src2d, dst2d, zeros)
    h, hn = _layer_mid(xn, p0, p1, x, Wl0, bl0r, Wr0, s0r)
    q0, q1 = _segsum_partials(hn, src2d, dst2d, zeros)
    out = _layer_final(hn, q0, q1, h, Wl1, bl1r, Wr1, s1r)
    return out


# speedup vs baseline: 1.2637x; 1.0245x over previous
import functools

import jax
import jax.numpy as jnp
from jax import lax
from jax.experimental import pallas as pl
from jax.experimental.pallas import tpu as pltpu
from jax.experimental.pallas import tpu_sc as plsc

N = 10000
D = 128
C = 40

NC = 2
NS = 16
BLK = 128
TBL = 10240
RPT = TBL // NS

_EPS = 1e-12


def _rownorm_kernel(x_ref, o_ref):
    v = x_ref[...]
    n = jnp.sqrt(jnp.sum(v * v, axis=1, keepdims=True))
    o_ref[...] = v / jnp.maximum(n, _EPS)


def _rownorm(x):
    rb = lambda i: (i, 0)
    return pl.pallas_call(
        _rownorm_kernel,
        grid=(N // 2000,),
        in_specs=[pl.BlockSpec((2000, D), rb)],
        out_specs=pl.BlockSpec((2000, D), rb),
        out_shape=jax.ShapeDtypeStruct((TBL, D), x.dtype),
    )(x)


def _segsum_partials(xnp, src2d, dst2d, zeros):
    eb = src2d.shape[0]
    bpt = eb // (NC * NS)
    mesh = plsc.VectorSubcoreMesh(core_axis_name="c", subcore_axis_name="s")

    @functools.partial(
        pl.kernel,
        out_type=(
            jax.ShapeDtypeStruct((TBL, D), jnp.float32),
            jax.ShapeDtypeStruct((TBL, D), jnp.float32),
            jax.ShapeDtypeStruct((eb * BLK, D), jnp.float32),
        ),
        mesh=mesh,
        scratch_types=[
            pltpu.VMEM((bpt, BLK), jnp.int32),
            pltpu.VMEM((BLK, D), jnp.float32),
            pltpu.VMEM((BLK, D), jnp.float32),
            pltpu.VMEM_SHARED((TBL, D), jnp.float32),
            pltpu.SemaphoreType.DMA,
            pltpu.SemaphoreType.DMA,
        ],
    )
    def k(xn_hbm, src_hbm, dst_hbm, z_hbm, out0_hbm, out1_hbm, msgs_hbm,
          idx_c, g0, g1, table, sem0, sem1):
        c = lax.axis_index("c")
        s = lax.axis_index("s")
        wid = s * NC + c
        base = wid * bpt

        pltpu.sync_copy(xn_hbm.at[pl.ds(s * RPT, RPT)],
                        table.at[pl.ds(s * RPT, RPT)])
        pltpu.sync_copy(src_hbm.at[pl.ds(base, bpt)], idx_c)
        plsc.subcore_barrier()
        pltpu.sync_copy(table.at[idx_c.at[0]], g0)

        @pl.loop(0, bpt // 2)
        def _(i):
            j = i * 2
            pltpu.async_copy(
                g0, msgs_hbm.at[pl.ds((base + j) * BLK, BLK)], sem0)
            pltpu.sync_copy(table.at[idx_c.at[j + 1]], g1)
            pltpu.make_async_copy(
                g0, msgs_hbm.at[pl.ds((base + j) * BLK, BLK)], sem0).wait()
            pltpu.async_copy(
                g1, msgs_hbm.at[pl.ds((base + j + 1) * BLK, BLK)], sem1)

            @pl.when(j + 2 < bpt)
            def _():
                pltpu.sync_copy(table.at[idx_c.at[j + 2]], g0)

            pltpu.make_async_copy(
                g1, msgs_hbm.at[pl.ds((base + j + 1) * BLK, BLK)],
                sem1).wait()

        plsc.subcore_barrier()

        pltpu.sync_copy(z_hbm, table.at[pl.ds(s * RPT, RPT)])
        pltpu.sync_copy(dst_hbm.at[pl.ds(base, bpt)], idx_c)
        plsc.subcore_barrier()
        pltpu.async_copy(msgs_hbm.at[pl.ds(base * BLK, BLK)], g0, sem0)

        @pl.loop(0, bpt // 2)
        def _(i):
            j = i * 2
            pltpu.make_async_copy(
                msgs_hbm.at[pl.ds((base + j) * BLK, BLK)], g0, sem0).wait()
            pltpu.async_copy(
                msgs_hbm.at[pl.ds((base + j + 1) * BLK, BLK)], g1, sem1)
            pltpu.sync_copy(g0, table.at[idx_c.at[j]], add=True)
            pltpu.make_async_copy(
                msgs_hbm.at[pl.ds((base + j + 1) * BLK, BLK)], g1,
                sem1).wait()

            @pl.when(j + 2 < bpt)
            def _():
                pltpu.async_copy(
                    msgs_hbm.at[pl.ds((base + j + 2) * BLK, BLK)], g0, sem0)

            pltpu.sync_copy(g1, table.at[idx_c.at[j + 1]], add=True)

        plsc.subcore_barrier()

        @pl.when(c == 0)
        def _():
            pltpu.sync_copy(table.at[pl.ds(s * RPT, RPT)],
                            out0_hbm.at[pl.ds(s * RPT, RPT)])

        @pl.when(c == 1)
        def _():
            pltpu.sync_copy(table.at[pl.ds(s * RPT, RPT)],
                            out1_hbm.at[pl.ds(s * RPT, RPT)])

    p0, p1, _ = k(xnp, src2d, dst2d, zeros)
    return p0, p1


def _dotT(a, w):
    return lax.dot_general(
        a, w, dimension_numbers=(((1,), (1,)), ((), ())),
        preferred_element_type=jnp.float32,
        precision=lax.Precision.HIGHEST)


def _layer_mid_kernel(xn_ref, p0_ref, p1_ref, x_ref, wl_ref, bl_ref, wr_ref,
                      sc_ref, h_ref, hn_ref):
    xn = xn_ref[...]
    x = x_ref[...]
    agg = xn + p0_ref[...] + p1_ref[...]
    nagg = jnp.sqrt(jnp.sum(agg * agg, axis=1, keepdims=True))
    aggn = agg / jnp.maximum(nagg, _EPS)
    xnorm = jnp.sqrt(jnp.sum(x * x, axis=1, keepdims=True))
    m = aggn * xnorm * sc_ref[0, 0]
    o = _dotT(m, wl_ref[...]) + bl_ref[...] + _dotT(x, wr_ref[...])
    no = jnp.sqrt(jnp.sum(o * o, axis=1, keepdims=True))
    h = jnp.maximum(o / jnp.maximum(no, _EPS), 0.0)
    h_ref[...] = h
    nh = jnp.sqrt(jnp.sum(h * h, axis=1, keepdims=True))
    hn_ref[...] = h / jnp.maximum(nh, _EPS)


def _layer_mid(xn, p0, p1, x, wl, bl, wr, sc):
    grid = (N // 2000,)
    rb = lambda i: (i, 0)
    wb = lambda i: (0, 0)
    return pl.pallas_call(
        _layer_mid_kernel,
        grid=grid,
        in_specs=[
            pl.BlockSpec((2000, D), rb),
            pl.BlockSpec((2000, D), rb),
            pl.BlockSpec((2000, D), rb),
            pl.BlockSpec((2000, D), rb),
            pl.BlockSpec((D, D), wb),
            pl.BlockSpec((1, D), wb),
            pl.BlockSpec((D, D), wb),
            pl.BlockSpec((1, 1), wb),
        ],
        out_specs=[
            pl.BlockSpec((2000, D), rb),
            pl.BlockSpec((2000, D), rb),
        ],
        out_shape=[
            jax.ShapeDtypeStruct((N, D), jnp.float32),
            jax.ShapeDtypeStruct((TBL, D), jnp.float32),
        ],
    )(xn, p0, p1, x, wl, bl, wr, sc)


def _layer_final_kernel(hn_ref, p0_ref, p1_ref, h_ref, wl_ref, bl_ref, wr_ref,
                        sc_ref, o_ref):
    hn = hn_ref[...]
    h = h_ref[...]
    agg = hn + p0_ref[...] + p1_ref[...]
    nagg = jnp.sqrt(jnp.sum(agg * agg, axis=1, keepdims=True))
    aggn = agg / jnp.maximum(nagg, _EPS)
    hnorm = jnp.sqrt(jnp.sum(h * h, axis=1, keepdims=True))
    m = aggn * hnorm * sc_ref[0, 0]
    o = _dotT(m, wl_ref[...]) + bl_ref[...] + _dotT(h, wr_ref[...])
    no = jnp.sqrt(jnp.sum(o * o, axis=1, keepdims=True))
    o_ref[...] = o / jnp.maximum(no, _EPS)


def _layer_final(hn, p0, p1, h, wl, bl, wr, sc):
    grid = (N // 2000,)
    rb = lambda i: (i, 0)
    wb = lambda i: (0, 0)
    return pl.pallas_call(
        _layer_final_kernel,
        grid=grid,
        in_specs=[
            pl.BlockSpec((2000, D), rb),
            pl.BlockSpec((2000, D), rb),
            pl.BlockSpec((2000, D), rb),
            pl.BlockSpec((2000, D), rb),
            pl.BlockSpec((C, D), wb),
            pl.BlockSpec((1, C), wb),
            pl.BlockSpec((C, D), wb),
            pl.BlockSpec((1, 1), wb),
        ],
        out_specs=pl.BlockSpec((2000, C), rb),
        out_shape=jax.ShapeDtypeStruct((N, C), jnp.float32),
    )(hn, p0, p1, h, wl, bl, wr, sc)


def kernel(x, edge_index, Wl0, bl0, Wr0, scale0, Wl1, bl1, Wr1, scale1):
    src = edge_index[0]
    dst = edge_index[1]
    e = src.shape[0]
    chunk = NC * NS * BLK * 8
    epad = ((e + chunk - 1) // chunk) * chunk
    pad = epad - e
    src_p = jnp.concatenate([src, jnp.zeros((pad,), jnp.int32)])
    dst_p = jnp.concatenate([dst, jnp.full((pad,), N, jnp.int32)])
    src2d = src_p.reshape(-1, BLK)
    dst2d = dst_p.reshape(-1, BLK)
    zeros = jnp.zeros((RPT, D), jnp.float32)
    bl0r = bl0.reshape(1, D)
    bl1r = bl1.reshape(1, C)
    s0r = scale0.reshape(1, 1)
    s1r = scale1.reshape(1, 1)

    xn = _rownorm(x)
    p0, p1 = _segsum_partials(xn, src2d, dst2d, zeros)
    h, hn = _layer_mid(xn, p0, p1, x, Wl0, bl0r, Wr0, s0r)
    q0, q1 = _segsum_partials(hn, src2d, dst2d, zeros)
    out = _layer_final(hn, q0, q1, h, Wl1, bl1r, Wr1, s1r)
    return out
